# Initial kernel scaffold; baseline (speedup 1.0000x reference)
import jax
import jax.numpy as jnp
from jax.experimental import pallas as pl


def kernel(x, edge_attr, glob, edge_index, batch, params):
    raise NotImplementedError("write your pallas kernel here")


# speedup vs baseline: 2.7077x; 2.7077x over previous
import functools

import jax
import jax.numpy as jnp
from jax import lax
from jax.experimental import pallas as pl
from jax.experimental.pallas import tpu as pltpu
from jax.experimental.pallas import tpu_sc as plsc

N = 10000
E = 320000
H = 64
D2 = 2 * H
B_G = 16
OUT_D = 201
OUT_P = 256

NC = 2
NS = 16
NW = NC * NS
CHUNK = 80
G_ITER = E // (NW * CHUNK)
EPC = E // NC
S_ITER = EPC // (NS * CHUNK)
ROWS_A = 624
ROWS_TAIL = N - (NS - 1) * ROWS_A


def _sc_gather_sum(a_tab, b_tab, ridx3, cidx3):
    mesh = plsc.VectorSubcoreMesh(core_axis_name="c", subcore_axis_name="s")

    @functools.partial(
        pl.kernel,
        mesh=mesh,
        out_type=jax.ShapeDtypeStruct((E, D2), jnp.float32),
        scratch_types=[
            pltpu.VMEM((G_ITER, CHUNK), jnp.int32),
            pltpu.VMEM((G_ITER, CHUNK), jnp.int32),
            pltpu.VMEM((CHUNK, D2), jnp.float32),
            pltpu.SemaphoreType.DMA,
            pltpu.SemaphoreType.DMA,
        ],
    )
    def k(a_hbm, b_hbm, ridx_hbm, cidx_hbm, out_hbm,
          ridx_v, cidx_v, buf, rsem, csem):
        wid = lax.axis_index("s") * NC + lax.axis_index("c")
        base = wid * (G_ITER * CHUNK)
        pltpu.sync_copy(ridx_hbm.at[wid], ridx_v)
        pltpu.sync_copy(cidx_hbm.at[wid], cidx_v)

        def body(j, carry):
            off = base + j * CHUNK
            pltpu.async_copy(a_hbm.at[ridx_v.at[j]], buf, rsem).wait()
            pltpu.async_copy(b_hbm.at[cidx_v.at[j]], buf, csem, add=True).wait()
            pltpu.sync_copy(buf, out_hbm.at[pl.ds(off, CHUNK)])
            return carry

        lax.fori_loop(0, G_ITER, body, 0)

    return k(a_tab, b_tab, ridx3, cidx3)


def _sc_scatter_add(new_e, cidx4, zeros_nh):
    mesh = plsc.VectorSubcoreMesh(core_axis_name="c", subcore_axis_name="s")

    @functools.partial(
        pl.kernel,
        mesh=mesh,
        out_type=jax.ShapeDtypeStruct((NC, N, D2), jnp.float32),
        scratch_types=[
            pltpu.VMEM((S_ITER, CHUNK), jnp.int32),
            pltpu.VMEM((CHUNK, D2), jnp.float32),
            pltpu.VMEM_SHARED((N, D2), jnp.float32),
        ],
    )
    def k(ne_hbm, cidx_hbm, zero_hbm, out_hbm, idx_v, ebuf, acc_sh):
        c = lax.axis_index("c")
        s = lax.axis_index("s")
        r0 = s * ROWS_A

        @pl.when(s < NS - 1)
        def _():
            pltpu.sync_copy(zero_hbm.at[pl.ds(r0, ROWS_A)],
                            acc_sh.at[pl.ds(r0, ROWS_A)])

        @pl.when(s == NS - 1)
        def _():
            pltpu.sync_copy(zero_hbm.at[pl.ds(r0, ROWS_TAIL)],
                            acc_sh.at[pl.ds(r0, ROWS_TAIL)])

        pltpu.sync_copy(cidx_hbm.at[c, s], idx_v)
        plsc.subcore_barrier()
        base = c * EPC + s * (S_ITER * CHUNK)

        def body(j, carry):
            off = base + j * CHUNK
            pltpu.sync_copy(ne_hbm.at[pl.ds(off, CHUNK)], ebuf)
            pltpu.sync_copy(ebuf, acc_sh.at[idx_v.at[j]], add=True)
            return carry

        lax.fori_loop(0, S_ITER, body, 0)
        plsc.subcore_barrier()

        @pl.when(s < NS - 1)
        def _():
            pltpu.sync_copy(acc_sh.at[pl.ds(r0, ROWS_A)],
                            out_hbm.at[c, pl.ds(r0, ROWS_A)])

        @pl.when(s == NS - 1)
        def _():
            pltpu.sync_copy(acc_sh.at[pl.ds(r0, ROWS_TAIL)],
                            out_hbm.at[c, pl.ds(r0, ROWS_TAIL)])

    return k(new_e, cidx4, zeros_nh)


def _enc_mlp(inp, w1, b1, av, w2, b2, bn):
    n, din = inp.shape
    dmid = w1.shape[1]
    dout = w2.shape[1]

    def body(x_ref, w1_ref, b1_ref, a_ref, w2_ref, b2_ref, o_ref):
        h = jnp.dot(x_ref[...], w1_ref[...],
                    preferred_element_type=jnp.float32) + b1_ref[...]
        h = jnp.where(h >= 0, h, a_ref[...] * h)
        o_ref[...] = jnp.dot(h, w2_ref[...],
                             preferred_element_type=jnp.float32) + b2_ref[...]

    return pl.pallas_call(
        body,
        grid=(n // bn,),
        in_specs=[
            pl.BlockSpec((bn, din), lambda i: (i, 0)),
            pl.BlockSpec((din, dmid), lambda i: (0, 0)),
            pl.BlockSpec((1, dmid), lambda i: (0, 0)),
            pl.BlockSpec((1, dmid), lambda i: (0, 0)),
            pl.BlockSpec((dmid, dout), lambda i: (0, 0)),
            pl.BlockSpec((1, dout), lambda i: (0, 0)),
        ],
        out_specs=pl.BlockSpec((bn, dout), lambda i: (i, 0)),
        out_shape=jax.ShapeDtypeStruct((n, dout), jnp.float32),
    )(inp, w1, b1.reshape(1, -1), av, w2, b2.reshape(1, -1))


def _proj_pair(xh, w1r, w1c):
    bn = 2000

    def body(x_ref, wr_ref, wc_ref, a_ref, b_ref):
        xb = x_ref[...]
        a_ref[...] = jnp.dot(xb, wr_ref[...], preferred_element_type=jnp.float32)
        b_ref[...] = jnp.dot(xb, wc_ref[...], preferred_element_type=jnp.float32)

    return pl.pallas_call(
        body,
        grid=(N // bn,),
        in_specs=[
            pl.BlockSpec((bn, H), lambda i: (i, 0)),
            pl.BlockSpec((H, D2), lambda i: (0, 0)),
            pl.BlockSpec((H, D2), lambda i: (0, 0)),
        ],
        out_specs=(pl.BlockSpec((bn, D2), lambda i: (i, 0)),
                   pl.BlockSpec((bn, D2), lambda i: (i, 0))),
        out_shape=(jax.ShapeDtypeStruct((N, D2), jnp.float32),
                   jax.ShapeDtypeStruct((N, D2), jnp.float32)),
    )(xh, w1r, w1c)


def _edge_mlp(h_rc, ea, w1e, b1, g, bb, av, w2, b2):
    be = 3200

    def body(h_ref, ea_ref, w1e_ref, b1_ref, g_ref, bb_ref, a_ref,
             w2_ref, b2_ref, ne_ref, e2_ref):
        eb = ea_ref[...]
        h = (h_ref[...]
             + jnp.dot(eb, w1e_ref[...], preferred_element_type=jnp.float32)
             + b1_ref[...])
        m = jnp.mean(h, axis=1, keepdims=True)
        v = jnp.mean(jnp.square(h - m), axis=1, keepdims=True)
        h = (h - m) * lax.rsqrt(v + 1e-5) * g_ref[...] + bb_ref[...]
        h = jnp.where(h >= 0, h, a_ref[...] * h)
        ne = jnp.dot(h, w2_ref[...], preferred_element_type=jnp.float32) + b2_ref[...]
        ne_ref[:, 0:H] = ne
        ne_ref[:, H:D2] = jnp.zeros((be, H), jnp.float32)
        e2_ref[...] = eb + ne

    return pl.pallas_call(
        body,
        grid=(E // be,),
        in_specs=[
            pl.BlockSpec((be, D2), lambda i: (i, 0)),
            pl.BlockSpec((be, H), lambda i: (i, 0)),
            pl.BlockSpec((H, D2), lambda i: (0, 0)),
            pl.BlockSpec((1, D2), lambda i: (0, 0)),
            pl.BlockSpec((1, D2), lambda i: (0, 0)),
            pl.BlockSpec((1, D2), lambda i: (0, 0)),
            pl.BlockSpec((1, D2), lambda i: (0, 0)),
            pl.BlockSpec((D2, H), lambda i: (0, 0)),
            pl.BlockSpec((1, H), lambda i: (0, 0)),
        ],
        out_specs=(pl.BlockSpec((be, D2), lambda i: (i, 0)),
                   pl.BlockSpec((be, H), lambda i: (i, 0))),
        out_shape=(jax.ShapeDtypeStruct((E, D2), jnp.float32),
                   jax.ShapeDtypeStruct((E, H), jnp.float32)),
    )(h_rc, ea, w1e, b1.reshape(1, -1), g.reshape(1, -1),
      bb.reshape(1, -1), av, w2, b2.reshape(1, -1))


def _node_mlp(x, parts, w1x, w1a, b1, g, bb, av, w2, b2):
    bn = 2000

    def body(x_ref, p_ref, w1x_ref, w1a_ref, b1_ref, g_ref, bb_ref, a_ref,
             w2_ref, b2_ref, xo_ref):
        xb = x_ref[...]
        agg = (p_ref[0] + p_ref[1])[:, 0:H]
        h = (jnp.dot(xb, w1x_ref[...], preferred_element_type=jnp.float32)
             + jnp.dot(agg, w1a_ref[...], preferred_element_type=jnp.float32)
             + b1_ref[...])
        m = jnp.mean(h, axis=1, keepdims=True)
        v = jnp.mean(jnp.square(h - m), axis=1, keepdims=True)
        h = (h - m) * lax.rsqrt(v + 1e-5) * g_ref[...] + bb_ref[...]
        h = jnp.where(h >= 0, h, a_ref[...] * h)
        nx = jnp.dot(h, w2_ref[...], preferred_element_type=jnp.float32) + b2_ref[...]
        xo_ref[...] = xb + nx

    return pl.pallas_call(
        body,
        grid=(N // bn,),
        in_specs=[
            pl.BlockSpec((bn, H), lambda i: (i, 0)),
            pl.BlockSpec((NC, bn, D2), lambda i: (0, i, 0)),
            pl.BlockSpec((H, D2), lambda i: (0, 0)),
            pl.BlockSpec((H, D2), lambda i: (0, 0)),
            pl.BlockSpec((1, D2), lambda i: (0, 0)),
            pl.BlockSpec((1, D2), lambda i: (0, 0)),
            pl.BlockSpec((1, D2), lambda i: (0, 0)),
            pl.BlockSpec((1, D2), lambda i: (0, 0)),
            pl.BlockSpec((D2, H), lambda i: (0, 0)),
            pl.BlockSpec((1, H), lambda i: (0, 0)),
        ],
        out_specs=pl.BlockSpec((bn, H), lambda i: (i, 0)),
        out_shape=jax.ShapeDtypeStruct((N, H), jnp.float32),
    )(x, parts, w1x, w1a, b1.reshape(1, -1), g.reshape(1, -1),
      bb.reshape(1, -1), av, w2, b2.reshape(1, -1))


def _out_head(x, batch2, w1, b1, g, bb, av, w2p, b2p):
    bn = 2000

    def body(x_ref, bt_ref, w1_ref, b1_ref, g_ref, bb_ref, a_ref,
             w2_ref, b2_ref, o_ref):
        i = pl.program_id(0)
        d = jnp.dot(x_ref[...], w1_ref[...],
                    preferred_element_type=jnp.float32) + b1_ref[...]
        m = jnp.mean(d, axis=1, keepdims=True)
        v = jnp.mean(jnp.square(d - m), axis=1, keepdims=True)
        d = (d - m) * lax.rsqrt(v + 1e-5) * g_ref[...] + bb_ref[...]
        d = jnp.where(d >= 0, d, a_ref[...] * d)
        dos = jnp.dot(d, w2_ref[...], preferred_element_type=jnp.float32) + b2_ref[...]
        onehot = (bt_ref[...] == lax.broadcasted_iota(jnp.int32, (1, B_G), 1)
                  ).astype(jnp.float32)
        contrib = lax.dot_general(onehot, dos, (((0,), (0,)), ((), ())),
                                  preferred_element_type=jnp.float32)

        @pl.when(i == 0)
        def _():
            o_ref[...] = contrib

        @pl.when(i != 0)
        def _():
            o_ref[...] += contrib

    return pl.pallas_call(
        body,
        grid=(N // bn,),
        in_specs=[
            pl.BlockSpec((bn, H), lambda i: (i, 0)),
            pl.BlockSpec((bn, 1), lambda i: (i, 0)),
            pl.BlockSpec((H, H), lambda i: (0, 0)),
            pl.BlockSpec((1, H), lambda i: (0, 0)),
            pl.BlockSpec((1, H), lambda i: (0, 0)),
            pl.BlockSpec((1, H), lambda i: (0, 0)),
            pl.BlockSpec((1, H), lambda i: (0, 0)),
            pl.BlockSpec((H, OUT_P), lambda i: (0, 0)),
            pl.BlockSpec((1, OUT_P), lambda i: (0, 0)),
        ],
        out_specs=pl.BlockSpec((B_G, OUT_P), lambda i: (0, 0)),
        out_shape=jax.ShapeDtypeStruct((B_G, OUT_P), jnp.float32),
    )(x, batch2, w1, b1.reshape(1, -1), g.reshape(1, -1), bb.reshape(1, -1),
      av, w2p, b2p)


def kernel(x, edge_attr, glob, edge_index, batch, params):
    p = params
    row = edge_index[0].astype(jnp.int32)
    col = edge_index[1].astype(jnp.int32)
    ridx3 = row.reshape(NW, G_ITER, CHUNK)
    cidx3 = col.reshape(NW, G_ITER, CHUNK)
    cidx4 = col.reshape(NC, NS, S_ITER, CHUNK)
    zeros_nh = jnp.zeros((N, D2), jnp.float32)
    batch2 = batch.astype(jnp.int32).reshape(N, 1)

    def bvec(a, d):
        return jnp.broadcast_to(jnp.asarray(a, jnp.float32).reshape(1, 1), (1, d))

    xh = _enc_mlp(x, p["enc_node_l1"]["W"], p["enc_node_l1"]["b"],
                  bvec(p["enc_node_a"], H),
                  p["enc_node_l2"]["W"], p["enc_node_l2"]["b"], 2000)
    eh = _enc_mlp(edge_attr, p["enc_edge_l1"]["W"], p["enc_edge_l1"]["b"],
                  bvec(p["enc_edge_a"], H),
                  p["enc_edge_l2"]["W"], p["enc_edge_l2"]["b"], 6400)

    for lp in p["proc"]:
        w1 = lp["edge_l1"]["W"]
        a_tab, b_tab = _proj_pair(xh, w1[0:H], w1[H:2 * H])
        h_rc = _sc_gather_sum(a_tab, b_tab, ridx3, cidx3)
        ne, e2 = _edge_mlp(h_rc, eh, w1[2 * H:3 * H],
                           lp["edge_l1"]["b"], lp["edge_ln_g"], lp["edge_ln_b"],
                           bvec(lp["edge_a"], D2),
                           lp["edge_l2"]["W"], lp["edge_l2"]["b"])
        parts = _sc_scatter_add(ne, cidx4, zeros_nh)
        nw1 = lp["node_l1"]["W"]
        xh = _node_mlp(xh, parts, nw1[0:H], nw1[H:2 * H], lp["node_l1"]["b"],
                       lp["node_ln_g"], lp["node_ln_b"], bvec(lp["node_a"], D2),
                       lp["node_l2"]["W"], lp["node_l2"]["b"])
        eh = e2

    w2p = jnp.pad(p["out_l2"]["W"], ((0, 0), (0, OUT_P - OUT_D)))
    b2p = jnp.pad(p["out_l2"]["b"], (0, OUT_P - OUT_D)).reshape(1, -1)
    dos3p = _out_head(xh, batch2, p["out_l1"]["W"], p["out_l1"]["b"],
                      p["out_ln_g"], p["out_ln_b"], bvec(p["out_a"], H),
                      w2p, b2p)
    return dos3p[:, :OUT_D], xh


# speedup vs baseline: 3.3258x; 1.2283x over previous
import functools

import jax
import jax.numpy as jnp
from jax import lax
from jax.experimental import pallas as pl
from jax.experimental.pallas import tpu as pltpu
from jax.experimental.pallas import tpu_sc as plsc

N = 10000
E = 320000
H = 64
D2 = 2 * H
B_G = 16
OUT_D = 201
OUT_P = 256

NC = 2
NS = 16
NW = NC * NS
CHUNK = 80
G_ITER = E // (NW * CHUNK)
EPC = E // NC
SCHUNK = 40
S_ITER = EPC // (NS * SCHUNK)
ROWS_A = 624
ROWS_TAIL = N - (NS - 1) * ROWS_A


NBUF = 5
SNBUF = 2


def _sc_gather_sum(a_tab, b_tab, ridx3, cidx3):
    mesh = plsc.VectorSubcoreMesh(core_axis_name="c", subcore_axis_name="s")

    @functools.partial(
        pl.kernel,
        mesh=mesh,
        out_type=jax.ShapeDtypeStruct((E, D2), jnp.float32),
        scratch_types=[
            pltpu.VMEM((G_ITER, CHUNK), jnp.int32),
            pltpu.VMEM((G_ITER, CHUNK), jnp.int32),
            pltpu.VMEM((NBUF, CHUNK, D2), jnp.float32),
        ] + [pltpu.SemaphoreType.DMA] * (3 * NBUF),
    )
    def k(a_hbm, b_hbm, ridx_hbm, cidx_hbm, out_hbm,
          ridx_v, cidx_v, bufs, *sems):
        sa = sems[0:NBUF]
        sb = sems[NBUF:2 * NBUF]
        ss = sems[2 * NBUF:3 * NBUF]
        wid = lax.axis_index("s") * NC + lax.axis_index("c")
        base = wid * (G_ITER * CHUNK)
        pltpu.sync_copy(ridx_hbm.at[wid], ridx_v)
        pltpu.sync_copy(cidx_hbm.at[wid], cidx_v)

        def body(i, carry):
            j0 = i * NBUF
            cpa = [pltpu.async_copy(a_hbm.at[ridx_v.at[j0 + kk]],
                                    bufs.at[kk], sa[kk])
                   for kk in range(NBUF)]
            cpb = []
            for kk in range(NBUF):
                cpa[kk].wait()
                cpb.append(pltpu.async_copy(b_hbm.at[cidx_v.at[j0 + kk]],
                                            bufs.at[kk], sb[kk], add=True))
            cps = []
            for kk in range(NBUF):
                cpb[kk].wait()
                off = base + (j0 + kk) * CHUNK
                cps.append(pltpu.async_copy(bufs.at[kk],
                                            out_hbm.at[pl.ds(off, CHUNK)],
                                            ss[kk]))
            for kk in range(NBUF):
                cps[kk].wait()
            return carry

        lax.fori_loop(0, G_ITER // NBUF, body, 0)

    return k(a_tab, b_tab, ridx3, cidx3)


def _sc_scatter_add(new_e, cidx4, zeros_nh):
    mesh = plsc.VectorSubcoreMesh(core_axis_name="c", subcore_axis_name="s")

    @functools.partial(
        pl.kernel,
        mesh=mesh,
        out_type=jax.ShapeDtypeStruct((NC, N, D2), jnp.float32),
        scratch_types=[
            pltpu.VMEM((S_ITER, SCHUNK), jnp.int32),
            pltpu.VMEM((SNBUF, SCHUNK, D2), jnp.float32),
            pltpu.VMEM_SHARED((N, D2), jnp.float32),
        ] + [pltpu.SemaphoreType.DMA] * SNBUF,
    )
    def k(ne_hbm, cidx_hbm, zero_hbm, out_hbm, idx_v, ebufs, acc_sh, *sems):
        sl = sems
        c = lax.axis_index("c")
        s = lax.axis_index("s")
        r0 = s * ROWS_A

        @pl.when(s < NS - 1)
        def _():
            pltpu.sync_copy(zero_hbm.at[pl.ds(r0, ROWS_A)],
                            acc_sh.at[pl.ds(r0, ROWS_A)])

        @pl.when(s == NS - 1)
        def _():
            pltpu.sync_copy(zero_hbm.at[pl.ds(r0, ROWS_TAIL)],
                            acc_sh.at[pl.ds(r0, ROWS_TAIL)])

        pltpu.sync_copy(cidx_hbm.at[c, s], idx_v)
        plsc.subcore_barrier()
        base = c * EPC + s * (S_ITER * SCHUNK)

        def body(i, carry):
            j0 = i * SNBUF
            cpl = [pltpu.async_copy(
                       ne_hbm.at[pl.ds(base + (j0 + kk) * SCHUNK, SCHUNK)],
                       ebufs.at[kk], sl[kk])
                   for kk in range(SNBUF)]
            for kk in range(SNBUF):
                cpl[kk].wait()
                pltpu.sync_copy(ebufs.at[kk], acc_sh.at[idx_v.at[j0 + kk]],
                                add=True)
            return carry

        lax.fori_loop(0, S_ITER // SNBUF, body, 0)
        plsc.subcore_barrier()

        @pl.when(s < NS - 1)
        def _():
            pltpu.sync_copy(acc_sh.at[pl.ds(r0, ROWS_A)],
                            out_hbm.at[c, pl.ds(r0, ROWS_A)])

        @pl.when(s == NS - 1)
        def _():
            pltpu.sync_copy(acc_sh.at[pl.ds(r0, ROWS_TAIL)],
                            out_hbm.at[c, pl.ds(r0, ROWS_TAIL)])

    return k(new_e, cidx4, zeros_nh)


def _enc_mlp(inp, w1, b1, av, w2, b2, bn):
    n, din = inp.shape
    dmid = w1.shape[1]
    dout = w2.shape[1]

    def body(x_ref, w1_ref, b1_ref, a_ref, w2_ref, b2_ref, o_ref):
        h = jnp.dot(x_ref[...], w1_ref[...],
                    preferred_element_type=jnp.float32) + b1_ref[...]
        h = jnp.where(h >= 0, h, a_ref[...] * h)
        o_ref[...] = jnp.dot(h, w2_ref[...],
                             preferred_element_type=jnp.float32) + b2_ref[...]

    return pl.pallas_call(
        body,
        grid=(n // bn,),
        in_specs=[
            pl.BlockSpec((bn, din), lambda i: (i, 0)),
            pl.BlockSpec((din, dmid), lambda i: (0, 0)),
            pl.BlockSpec((1, dmid), lambda i: (0, 0)),
            pl.BlockSpec((1, dmid), lambda i: (0, 0)),
            pl.BlockSpec((dmid, dout), lambda i: (0, 0)),
            pl.BlockSpec((1, dout), lambda i: (0, 0)),
        ],
        out_specs=pl.BlockSpec((bn, dout), lambda i: (i, 0)),
        out_shape=jax.ShapeDtypeStruct((n, dout), jnp.float32),
    )(inp, w1, b1.reshape(1, -1), av, w2, b2.reshape(1, -1))


def _proj_pair(xh, w1r, w1c):
    bn = 2000

    def body(x_ref, wr_ref, wc_ref, a_ref, b_ref):
        xb = x_ref[...]
        a_ref[...] = jnp.dot(xb, wr_ref[...], preferred_element_type=jnp.float32)
        b_ref[...] = jnp.dot(xb, wc_ref[...], preferred_element_type=jnp.float32)

    return pl.pallas_call(
        body,
        grid=(N // bn,),
        in_specs=[
            pl.BlockSpec((bn, H), lambda i: (i, 0)),
            pl.BlockSpec((H, D2), lambda i: (0, 0)),
            pl.BlockSpec((H, D2), lambda i: (0, 0)),
        ],
        out_specs=(pl.BlockSpec((bn, D2), lambda i: (i, 0)),
                   pl.BlockSpec((bn, D2), lambda i: (i, 0))),
        out_shape=(jax.ShapeDtypeStruct((N, D2), jnp.float32),
                   jax.ShapeDtypeStruct((N, D2), jnp.float32)),
    )(xh, w1r, w1c)


def _edge_mlp(h_rc, ea, w1e, b1, g, bb, av, w2, b2):
    be = 3200

    def body(h_ref, ea_ref, w1e_ref, b1_ref, g_ref, bb_ref, a_ref,
             w2_ref, b2_ref, ne_ref, e2_ref):
        eb = ea_ref[...]
        h = (h_ref[...]
             + jnp.dot(eb, w1e_ref[...], preferred_element_type=jnp.float32)
             + b1_ref[...])
        m = jnp.mean(h, axis=1, keepdims=True)
        v = jnp.mean(jnp.square(h - m), axis=1, keepdims=True)
        h = (h - m) * lax.rsqrt(v + 1e-5) * g_ref[...] + bb_ref[...]
        h = jnp.where(h >= 0, h, a_ref[...] * h)
        ne = jnp.dot(h, w2_ref[...], preferred_element_type=jnp.float32) + b2_ref[...]
        ne_ref[:, 0:H] = ne
        ne_ref[:, H:D2] = jnp.zeros((be, H), jnp.float32)
        e2_ref[...] = eb + ne

    return pl.pallas_call(
        body,
        grid=(E // be,),
        in_specs=[
            pl.BlockSpec((be, D2), lambda i: (i, 0)),
            pl.BlockSpec((be, H), lambda i: (i, 0)),
            pl.BlockSpec((H, D2), lambda i: (0, 0)),
            pl.BlockSpec((1, D2), lambda i: (0, 0)),
            pl.BlockSpec((1, D2), lambda i: (0, 0)),
            pl.BlockSpec((1, D2), lambda i: (0, 0)),
            pl.BlockSpec((1, D2), lambda i: (0, 0)),
            pl.BlockSpec((D2, H), lambda i: (0, 0)),
            pl.BlockSpec((1, H), lambda i: (0, 0)),
        ],
        out_specs=(pl.BlockSpec((be, D2), lambda i: (i, 0)),
                   pl.BlockSpec((be, H), lambda i: (i, 0))),
        out_shape=(jax.ShapeDtypeStruct((E, D2), jnp.float32),
                   jax.ShapeDtypeStruct((E, H), jnp.float32)),
    )(h_rc, ea, w1e, b1.reshape(1, -1), g.reshape(1, -1),
      bb.reshape(1, -1), av, w2, b2.reshape(1, -1))


def _node_mlp(x, parts, w1x, w1a, b1, g, bb, av, w2, b2):
    bn = 2000

    def body(x_ref, p_ref, w1x_ref, w1a_ref, b1_ref, g_ref, bb_ref, a_ref,
             w2_ref, b2_ref, xo_ref):
        xb = x_ref[...]
        agg = (p_ref[0] + p_ref[1])[:, 0:H]
        h = (jnp.dot(xb, w1x_ref[...], preferred_element_type=jnp.float32)
             + jnp.dot(agg, w1a_ref[...], preferred_element_type=jnp.float32)
             + b1_ref[...])
        m = jnp.mean(h, axis=1, keepdims=True)
        v = jnp.mean(jnp.square(h - m), axis=1, keepdims=True)
        h = (h - m) * lax.rsqrt(v + 1e-5) * g_ref[...] + bb_ref[...]
        h = jnp.where(h >= 0, h, a_ref[...] * h)
        nx = jnp.dot(h, w2_ref[...], preferred_element_type=jnp.float32) + b2_ref[...]
        xo_ref[...] = xb + nx

    return pl.pallas_call(
        body,
        grid=(N // bn,),
        in_specs=[
            pl.BlockSpec((bn, H), lambda i: (i, 0)),
            pl.BlockSpec((NC, bn, D2), lambda i: (0, i, 0)),
            pl.BlockSpec((H, D2), lambda i: (0, 0)),
            pl.BlockSpec((H, D2), lambda i: (0, 0)),
            pl.BlockSpec((1, D2), lambda i: (0, 0)),
            pl.BlockSpec((1, D2), lambda i: (0, 0)),
            pl.BlockSpec((1, D2), lambda i: (0, 0)),
            pl.BlockSpec((1, D2), lambda i: (0, 0)),
            pl.BlockSpec((D2, H), lambda i: (0, 0)),
            pl.BlockSpec((1, H), lambda i: (0, 0)),
        ],
        out_specs=pl.BlockSpec((bn, H), lambda i: (i, 0)),
        out_shape=jax.ShapeDtypeStruct((N, H), jnp.float32),
    )(x, parts, w1x, w1a, b1.reshape(1, -1), g.reshape(1, -1),
      bb.reshape(1, -1), av, w2, b2.reshape(1, -1))


def _out_head(x, batch2, w1, b1, g, bb, av, w2p, b2p):
    bn = 2000

    def body(x_ref, bt_ref, w1_ref, b1_ref, g_ref, bb_ref, a_ref,
             w2_ref, b2_ref, o_ref):
        i = pl.program_id(0)
        d = jnp.dot(x_ref[...], w1_ref[...],
                    preferred_element_type=jnp.float32) + b1_ref[...]
        m = jnp.mean(d, axis=1, keepdims=True)
        v = jnp.mean(jnp.square(d - m), axis=1, keepdims=True)
        d = (d - m) * lax.rsqrt(v + 1e-5) * g_ref[...] + bb_ref[...]
        d = jnp.where(d >= 0, d, a_ref[...] * d)
        dos = jnp.dot(d, w2_ref[...], preferred_element_type=jnp.float32) + b2_ref[...]
        onehot = (bt_ref[...] == lax.broadcasted_iota(jnp.int32, (1, B_G), 1)
                  ).astype(jnp.float32)
        contrib = lax.dot_general(onehot, dos, (((0,), (0,)), ((), ())),
                                  preferred_element_type=jnp.float32)

        @pl.when(i == 0)
        def _():
            o_ref[...] = contrib

        @pl.when(i != 0)
        def _():
            o_ref[...] += contrib

    return pl.pallas_call(
        body,
        grid=(N // bn,),
        in_specs=[
            pl.BlockSpec((bn, H), lambda i: (i, 0)),
            pl.BlockSpec((bn, 1), lambda i: (i, 0)),
            pl.BlockSpec((H, H), lambda i: (0, 0)),
            pl.BlockSpec((1, H), lambda i: (0, 0)),
            pl.BlockSpec((1, H), lambda i: (0, 0)),
            pl.BlockSpec((1, H), lambda i: (0, 0)),
            pl.BlockSpec((1, H), lambda i: (0, 0)),
            pl.BlockSpec((H, OUT_P), lambda i: (0, 0)),
            pl.BlockSpec((1, OUT_P), lambda i: (0, 0)),
        ],
        out_specs=pl.BlockSpec((B_G, OUT_P), lambda i: (0, 0)),
        out_shape=jax.ShapeDtypeStruct((B_G, OUT_P), jnp.float32),
    )(x, batch2, w1, b1.reshape(1, -1), g.reshape(1, -1), bb.reshape(1, -1),
      av, w2p, b2p)


def kernel(x, edge_attr, glob, edge_index, batch, params):
    p = params
    row = edge_index[0].astype(jnp.int32)
    col = edge_index[1].astype(jnp.int32)
    ridx3 = row.reshape(NW, G_ITER, CHUNK)
    cidx3 = col.reshape(NW, G_ITER, CHUNK)
    cidx4 = col.reshape(NC, NS, S_ITER, SCHUNK)
    zeros_nh = jnp.zeros((N, D2), jnp.float32)
    batch2 = batch.astype(jnp.int32).reshape(N, 1)

    def bvec(a, d):
        return jnp.broadcast_to(jnp.asarray(a, jnp.float32).reshape(1, 1), (1, d))

    xh = _enc_mlp(x, p["enc_node_l1"]["W"], p["enc_node_l1"]["b"],
                  bvec(p["enc_node_a"], H),
                  p["enc_node_l2"]["W"], p["enc_node_l2"]["b"], 2000)
    eh = _enc_mlp(edge_attr, p["enc_edge_l1"]["W"], p["enc_edge_l1"]["b"],
                  bvec(p["enc_edge_a"], H),
                  p["enc_edge_l2"]["W"], p["enc_edge_l2"]["b"], 6400)

    for lp in p["proc"]:
        w1 = lp["edge_l1"]["W"]
        a_tab, b_tab = _proj_pair(xh, w1[0:H], w1[H:2 * H])
        h_rc = _sc_gather_sum(a_tab, b_tab, ridx3, cidx3)
        ne, e2 = _edge_mlp(h_rc, eh, w1[2 * H:3 * H],
                           lp["edge_l1"]["b"], lp["edge_ln_g"], lp["edge_ln_b"],
                           bvec(lp["edge_a"], D2),
                           lp["edge_l2"]["W"], lp["edge_l2"]["b"])
        parts = _sc_scatter_add(ne, cidx4, zeros_nh)
        nw1 = lp["node_l1"]["W"]
        xh = _node_mlp(xh, parts, nw1[0:H], nw1[H:2 * H], lp["node_l1"]["b"],
                       lp["node_ln_g"], lp["node_ln_b"], bvec(lp["node_a"], D2),
                       lp["node_l2"]["W"], lp["node_l2"]["b"])
        eh = e2

    w2p = jnp.pad(p["out_l2"]["W"], ((0, 0), (0, OUT_P - OUT_D)))
    b2p = jnp.pad(p["out_l2"]["b"], (0, OUT_P - OUT_D)).reshape(1, -1)
    dos3p = _out_head(xh, batch2, p["out_l1"]["W"], p["out_l1"]["b"],
                      p["out_ln_g"], p["out_ln_b"], bvec(p["out_a"], H),
                      w2p, b2p)
    return dos3p[:, :OUT_D], xh
